# baseline (speedup 1.0000x reference)
import jax
import jax.numpy as jnp
from jax.experimental import pallas as pl

N_CLASS = 500
N_DIM = 10
BATCH = 16384
GRID = 8
BLOCK = BATCH // GRID


def _fused_kernel(x_ref, k_ref, w_ref, b_ref, out_ref, loss_ref):
    k = k_ref[...]
    w = w_ref[...]
    w_eff = jnp.sum(k * w[:, 0][None, :], axis=1)
    x = x_ref[...]
    out_ref[...] = jnp.sum(x * w_eff[None, :], axis=1, keepdims=True) + b_ref[...]

    @pl.when(pl.program_id(0) == 0)
    def _():
        mean = jnp.mean(k, axis=1, keepdims=True)
        cent = k - mean
        loss_ref[...] = (jnp.sum(cent * cent) / (N_CLASS - 1.0)).reshape(1, 1)


def kernel(inputs, dense_cov_kernel, final_w, final_b, segment_ids):
    del segment_ids
    b = final_b.reshape(1, 1)
    out, loss = pl.pallas_call(
        _fused_kernel,
        grid=(GRID,),
        in_specs=[
            pl.BlockSpec((BLOCK, N_DIM), lambda i: (i, 0)),
            pl.BlockSpec((N_DIM, N_CLASS), lambda i: (0, 0)),
            pl.BlockSpec((N_CLASS, 1), lambda i: (0, 0)),
            pl.BlockSpec((1, 1), lambda i: (0, 0)),
        ],
        out_specs=(
            pl.BlockSpec((BLOCK, 1), lambda i: (i, 0)),
            pl.BlockSpec((1, 1), lambda i: (0, 0)),
        ),
        out_shape=(
            jax.ShapeDtypeStruct((BATCH, 1), jnp.float32),
            jax.ShapeDtypeStruct((1, 1), jnp.float32),
        ),
    )(inputs, dense_cov_kernel, final_w, b)
    return out, loss[0, 0]


# speedup vs baseline: 1.4437x; 1.4437x over previous
import jax
import jax.numpy as jnp
from jax.experimental import pallas as pl
from jax.experimental.pallas import tpu as pltpu

N_CLASS = 500
N_DIM = 10
BATCH = 16384
ROWS = 128
COLS = 128
LANES = COLS * N_DIM


def _fused_kernel(x_ref, k_ref, w_ref, b_ref, out_ref, loss_ref):
    k = k_ref[...]
    w = w_ref[...]
    w_eff = jnp.dot(k, w, preferred_element_type=jnp.float32)
    l_col = jax.lax.broadcasted_iota(jnp.int32, (LANES, N_DIM), 0)
    d_col = jax.lax.broadcasted_iota(jnp.int32, (LANES, N_DIM), 1)
    t_onehot = (jax.lax.rem(l_col, N_DIM) == d_col).astype(jnp.float32)
    w_pat = jnp.dot(t_onehot, w_eff, preferred_element_type=jnp.float32)
    l_iota = jax.lax.broadcasted_iota(jnp.int32, (LANES, COLS), 0)
    j_iota = jax.lax.broadcasted_iota(jnp.int32, (LANES, COLS), 1)
    j10 = j_iota * N_DIM
    s_band = ((l_iota >= j10) & (l_iota < j10 + N_DIM)).astype(jnp.float32)
    m = s_band * w_pat
    x = x_ref[...]
    out_ref[...] = (
        jnp.dot(x, m, preferred_element_type=jnp.float32) + b_ref[0]
    )
    mean = jnp.mean(k, axis=1, keepdims=True)
    cent = k - mean
    loss_ref[...] = (jnp.sum(cent * cent) / (N_CLASS - 1.0)).reshape(1, 1)


def kernel(inputs, dense_cov_kernel, final_w, final_b, segment_ids):
    del segment_ids
    x = inputs.reshape(ROWS, LANES)
    out, loss = pl.pallas_call(
        _fused_kernel,
        in_specs=[
            pl.BlockSpec((ROWS, LANES), lambda: (0, 0)),
            pl.BlockSpec((N_DIM, N_CLASS), lambda: (0, 0)),
            pl.BlockSpec((N_CLASS, 1), lambda: (0, 0)),
            pl.BlockSpec(memory_space=pltpu.SMEM),
        ],
        out_shape=(
            jax.ShapeDtypeStruct((ROWS, COLS), jnp.float32),
            jax.ShapeDtypeStruct((1, 1), jnp.float32),
        ),
    )(x, dense_cov_kernel, final_w, final_b)
    return out.reshape(BATCH, 1), loss[0, 0]
